# baseline (speedup 1.0000x reference)
import functools
import jax
import jax.numpy as jnp
from jax import lax
from jax.experimental import pallas as pl
from jax.experimental.pallas import tpu as pltpu
from jax.experimental.pallas import tpu_sc as plsc

D = 64
B = 1024
L = 200
VOCAB = 100000

LP = 208
NCHUNK = LP // 16
ZCH = 4096
HIST_N = 102400
NWORKERS = 32
ROWS_PER_W = B // NWORKERS


def _sc_counts(src, dst):
    mesh = plsc.VectorSubcoreMesh(core_axis_name="c", subcore_axis_name="s")
    f32 = jnp.float32

    @functools.partial(
        pl.kernel,
        mesh=mesh,
        out_type=[jax.ShapeDtypeStruct((B * L,), f32) for _ in range(4)],
        scratch_types=[
            pltpu.VMEM_SHARED((16 * HIST_N,), f32),
            pltpu.VMEM((LP,), jnp.int32),
            pltpu.VMEM((LP,), jnp.int32),
            pltpu.VMEM((LP,), f32),
            pltpu.VMEM((LP,), f32),
            pltpu.VMEM((ZCH,), f32),
            pltpu.VMEM((LP,), f32),
            pltpu.VMEM((LP,), f32),
            pltpu.VMEM((LP,), f32),
            pltpu.VMEM((LP,), f32),
        ],
    )
    def sc_kernel(src_hbm, dst_hbm, css_hbm, csd_hbm, cds_hbm, cdd_hbm,
                  hist, srow, drow, ones, zeros, zbig, css, csd, cds, cdd):
        cid = lax.axis_index("c")
        sid = lax.axis_index("s")
        wid = sid * 2 + cid
        base = wid * ROWS_PER_W
        bias = sid * HIST_N

        zero16 = jnp.zeros((16,), f32)
        one16 = jnp.ones((16,), f32)
        iota16 = lax.iota(jnp.int32, 16)

        def zinit(i, carry):
            off = pl.multiple_of(i * 256, 256)
            for j in range(16):
                zbig[pl.ds(off + j * 16, 16)] = zero16
            return carry
        lax.fori_loop(0, ZCH // 256, zinit, 0)
        for c in range(NCHUNK):
            ones[pl.ds(c * 16, 16)] = one16
            zeros[pl.ds(c * 16, 16)] = zero16
        for k in range(HIST_N // ZCH):
            pltpu.sync_copy(zbig, hist.at[pl.ds(bias + k * ZCH, ZCH)])

        def load_row(hbm, buf, b):
            pltpu.sync_copy(hbm.at[pl.ds(b * L, L)], buf.at[pl.ds(0, L)])
            for c in range(NCHUNK):
                v = buf[pl.ds(c * 16, 16)]
                if c == NCHUNK - 1:
                    v = jnp.where(iota16 < L - (NCHUNK - 1) * 16, v, 0)
                buf[pl.ds(c * 16, 16)] = v + bias

        def gather_counts(idbuf, out_buf):
            pltpu.sync_copy(hist.at[idbuf], out_buf)
            for c in range(NCHUNK):
                idx = idbuf[pl.ds(c * 16, 16)]
                g = out_buf[pl.ds(c * 16, 16)]
                out_buf[pl.ds(c * 16, 16)] = jnp.where(idx == bias, 0.0, g)

        def row_body(i, carry):
            b = base + i
            load_row(src_hbm, srow, b)
            load_row(dst_hbm, drow, b)

            pltpu.sync_copy(ones, hist.at[srow], add=True)
            gather_counts(srow, css)
            gather_counts(drow, cds)
            pltpu.sync_copy(zeros, hist.at[srow])

            pltpu.sync_copy(ones, hist.at[drow], add=True)
            gather_counts(srow, csd)
            gather_counts(drow, cdd)
            pltpu.sync_copy(zeros, hist.at[drow])

            off = b * L
            pltpu.sync_copy(css.at[pl.ds(0, L)], css_hbm.at[pl.ds(off, L)])
            pltpu.sync_copy(csd.at[pl.ds(0, L)], csd_hbm.at[pl.ds(off, L)])
            pltpu.sync_copy(cds.at[pl.ds(0, L)], cds_hbm.at[pl.ds(off, L)])
            pltpu.sync_copy(cdd.at[pl.ds(0, L)], cdd_hbm.at[pl.ds(off, L)])
            return carry

        lax.fori_loop(0, ROWS_PER_W, row_body, 0)

    return sc_kernel(src.reshape(B * L), dst.reshape(B * L))


def _tc_encode(css, csd, cds, cdd, w1r, b1r, w2t, b2r):
    P = 512
    grid = (B * L) // P

    def body(css_ref, csd_ref, cds_ref, cdd_ref, w1_ref, b1_ref, w2t_ref,
             b2_ref, osrc_ref, odst_ref):
        w1 = w1_ref[...]
        b1 = b1_ref[...]
        w2t = w2t_ref[...]
        b2 = b2_ref[...]

        def enc(c0, c1):
            h = jnp.maximum(c0 * w1 + b1, 0.0) + jnp.maximum(c1 * w1 + b1, 0.0)
            return jnp.dot(h, w2t, preferred_element_type=jnp.float32) + 2.0 * b2

        osrc_ref[...] = enc(css_ref[...], csd_ref[...])
        odst_ref[...] = enc(cds_ref[...], cdd_ref[...])

    cnt_spec = pl.BlockSpec((P, 1), lambda i: (i, 0))
    w_spec = pl.BlockSpec((1, D), lambda i: (0, 0))
    w2_spec = pl.BlockSpec((D, D), lambda i: (0, 0))
    out_spec = pl.BlockSpec((P, D), lambda i: (i, 0))

    return pl.pallas_call(
        body,
        grid=grid,
        in_specs=[cnt_spec, cnt_spec, cnt_spec, cnt_spec,
                  w_spec, w_spec, w2_spec, w_spec],
        out_specs=[out_spec, out_spec],
        out_shape=[jax.ShapeDtypeStruct((B * L, D), jnp.float32)] * 2,
    )(css, csd, cds, cdd, w1r, b1r, w2t, b2r)


@jax.jit
def kernel(src_padded_nodes_neighbor_ids, dst_padded_nodes_neighbor_ids,
           W1, b1, W2, b2):
    src = src_padded_nodes_neighbor_ids
    dst = dst_padded_nodes_neighbor_ids

    c_ss, c_sd, c_ds, c_dd = _sc_counts(src, dst)

    w1r = W1.reshape(1, D)
    b1r = b1.reshape(1, D)
    w2t = W2.T
    b2r = b2.reshape(1, D)

    out_src, out_dst = _tc_encode(
        c_ss.reshape(B * L, 1), c_sd.reshape(B * L, 1),
        c_ds.reshape(B * L, 1), c_dd.reshape(B * L, 1),
        w1r, b1r, w2t, b2r)

    return out_src.reshape(B, L, D), out_dst.reshape(B, L, D)


# speedup vs baseline: 2.4463x; 2.4463x over previous
import functools
import jax
import jax.numpy as jnp
from jax import lax
from jax.experimental import pallas as pl
from jax.experimental.pallas import tpu as pltpu
from jax.experimental.pallas import tpu_sc as plsc

D = 64
B = 1024
L = 200
VOCAB = 100000

LP = 208
NCHUNK = LP // 16
ZCH = 4096
HIST_N = 102400
NWORKERS = 32
ROWS_PER_W = B // NWORKERS
CN = 2048
CROWS = 104
PADN = CROWS * CN


def _sc_counts(src, dst):
    mesh = plsc.VectorSubcoreMesh(core_axis_name="c", subcore_axis_name="s")
    f32 = jnp.float32

    @functools.partial(
        pl.kernel,
        mesh=mesh,
        out_type=[jax.ShapeDtypeStruct((PADN,), f32) for _ in range(4)],
        scratch_types=[
            pltpu.VMEM_SHARED((16 * HIST_N,), f32),
            pltpu.VMEM((LP,), jnp.int32),
            pltpu.VMEM((LP,), jnp.int32),
            pltpu.VMEM((LP,), f32),
            pltpu.VMEM((LP,), f32),
            pltpu.VMEM((ZCH,), f32),
            pltpu.VMEM((LP,), f32),
            pltpu.VMEM((LP,), f32),
            pltpu.VMEM((LP,), f32),
            pltpu.VMEM((LP,), f32),
        ],
    )
    def sc_kernel(src_hbm, dst_hbm, css_hbm, csd_hbm, cds_hbm, cdd_hbm,
                  hist, srow, drow, ones, zeros, zbig, css, csd, cds, cdd):
        cid = lax.axis_index("c")
        sid = lax.axis_index("s")
        wid = sid * 2 + cid
        base = wid * ROWS_PER_W
        bias = sid * HIST_N

        zero16 = jnp.zeros((16,), f32)
        one16 = jnp.ones((16,), f32)
        iota16 = lax.iota(jnp.int32, 16)

        def zinit(i, carry):
            off = pl.multiple_of(i * 256, 256)
            for j in range(16):
                zbig[pl.ds(off + j * 16, 16)] = zero16
            return carry
        lax.fori_loop(0, ZCH // 256, zinit, 0)
        for c in range(NCHUNK):
            ones[pl.ds(c * 16, 16)] = one16
            zeros[pl.ds(c * 16, 16)] = zero16
        for k in range(HIST_N // ZCH):
            pltpu.sync_copy(zbig, hist.at[pl.ds(bias + k * ZCH, ZCH)])

        def load_row(hbm, buf, b):
            pltpu.sync_copy(hbm.at[pl.ds(b * L, L)], buf.at[pl.ds(0, L)])
            for c in range(NCHUNK):
                v = buf[pl.ds(c * 16, 16)]
                if c == NCHUNK - 1:
                    v = jnp.where(iota16 < L - (NCHUNK - 1) * 16, v, 0)
                buf[pl.ds(c * 16, 16)] = v + bias

        def gather_counts(idbuf, out_buf):
            pltpu.sync_copy(hist.at[idbuf], out_buf)
            for c in range(NCHUNK):
                idx = idbuf[pl.ds(c * 16, 16)]
                g = out_buf[pl.ds(c * 16, 16)]
                out_buf[pl.ds(c * 16, 16)] = jnp.where(idx == bias, 0.0, g)

        def row_body(i, carry):
            b = base + i
            load_row(src_hbm, srow, b)
            load_row(dst_hbm, drow, b)

            pltpu.sync_copy(ones, hist.at[srow], add=True)
            gather_counts(srow, css)
            gather_counts(drow, cds)
            pltpu.sync_copy(zeros, hist.at[srow])

            pltpu.sync_copy(ones, hist.at[drow], add=True)
            gather_counts(srow, csd)
            gather_counts(drow, cdd)
            pltpu.sync_copy(zeros, hist.at[drow])

            off = b * L
            pltpu.sync_copy(css.at[pl.ds(0, L)], css_hbm.at[pl.ds(off, L)])
            pltpu.sync_copy(csd.at[pl.ds(0, L)], csd_hbm.at[pl.ds(off, L)])
            pltpu.sync_copy(cds.at[pl.ds(0, L)], cds_hbm.at[pl.ds(off, L)])
            pltpu.sync_copy(cdd.at[pl.ds(0, L)], cdd_hbm.at[pl.ds(off, L)])
            return carry

        lax.fori_loop(0, ROWS_PER_W, row_body, 0)

    return sc_kernel(src.reshape(B * L), dst.reshape(B * L))


def _tc_encode(css, csd, cds, cdd, w1c, b1c, w2, b2c):
    grid = ((B * L) // CN,)

    def body(css_ref, csd_ref, cds_ref, cdd_ref, w1_ref, b1_ref, w2_ref,
             b2_ref, osrc_ref, odst_ref):
        j = pl.program_id(0) % 8
        w1 = w1_ref[...]
        b1 = b1_ref[...]
        w2 = w2_ref[...]
        b2 = b2_ref[...]

        def enc(c_ref0, c_ref1, out_ref):
            c0 = c_ref0[pl.ds(j, 1), :]
            c1 = c_ref1[pl.ds(j, 1), :]
            h = (jnp.maximum(c0 * w1 + b1, 0.0)
                 + jnp.maximum(c1 * w1 + b1, 0.0))
            out_t = jnp.dot(w2, h, preferred_element_type=jnp.float32) \
                + 2.0 * b2
            out_ref[...] = out_t.T

        enc(css_ref, csd_ref, osrc_ref)
        enc(cds_ref, cdd_ref, odst_ref)

    cnt_spec = pl.BlockSpec((8, CN), lambda i: (i // 8, 0))
    wcol_spec = pl.BlockSpec((D, 1), lambda i: (0, 0))
    w2_spec = pl.BlockSpec((D, D), lambda i: (0, 0))
    out_spec = pl.BlockSpec((CN, D), lambda i: (i, 0))

    return pl.pallas_call(
        body,
        grid=grid,
        in_specs=[cnt_spec, cnt_spec, cnt_spec, cnt_spec,
                  wcol_spec, wcol_spec, w2_spec, wcol_spec],
        out_specs=[out_spec, out_spec],
        out_shape=[jax.ShapeDtypeStruct((B * L, D), jnp.float32)] * 2,
    )(css, csd, cds, cdd, w1c, b1c, w2, b2c)


@jax.jit
def kernel(src_padded_nodes_neighbor_ids, dst_padded_nodes_neighbor_ids,
           W1, b1, W2, b2):
    src = src_padded_nodes_neighbor_ids
    dst = dst_padded_nodes_neighbor_ids

    c_ss, c_sd, c_ds, c_dd = _sc_counts(src, dst)

    b1c = b1.reshape(D, 1)
    b2c = b2.reshape(D, 1)

    out_src, out_dst = _tc_encode(
        c_ss.reshape(CROWS, CN), c_sd.reshape(CROWS, CN),
        c_ds.reshape(CROWS, CN), c_dd.reshape(CROWS, CN),
        W1, b1c, W2, b2c)

    return out_src.reshape(B, L, D), out_dst.reshape(B, L, D)
